# baseline (speedup 1.0000x reference)
import functools

import jax
import jax.numpy as jnp
from jax import lax
from jax.experimental import pallas as pl
from jax.experimental.pallas import tpu as pltpu
from jax.experimental.pallas import tpu_sc as plsc

NU = 50000
NI = 50000
NN = NU + NI
NNP = 100096
P = NNP // 8
EDG = 1600000
K = 32
KH = 16
NL = 3

NC = 2
NS = 16

CH = 400
PER_TEC = EDG // NS
N_CH = PER_TEC // CH
ROWS_PER_TEC = NNP // NS

CH_D = 1000
PER_TEC_D = EDG // (NC * NS)
N_CH_D = PER_TEC_D // CH_D

_mesh = plsc.VectorSubcoreMesh(core_axis_name="c", subcore_axis_name="s")


@functools.partial(
    pl.kernel,
    out_type=jax.ShapeDtypeStruct((NC, NNP, KH), jnp.float32),
    mesh=_mesh,
    scratch_types=[
        pltpu.VMEM((4, CH), jnp.int32),
        pltpu.VMEM((4, CH), jnp.int32),
        pltpu.VMEM((3, CH, KH), jnp.float32),
        pltpu.VMEM_SHARED((NNP, KH), jnp.float32),
        pltpu.SemaphoreType.DMA,
        pltpu.SemaphoreType.DMA((3,)),
        pltpu.SemaphoreType.DMA((2,)),
    ],
    compiler_params=pltpu.CompilerParams(use_tc_tiling_on_sc=False),
)
def _sc_spmm(xcat, coli, rowi, out, colv, rowv, gat, acc, semi, semg, sems):
    c = lax.axis_index("c")
    s = lax.axis_index("s")
    slab = pl.ds(s * ROWS_PER_TEC, ROWS_PER_TEC)

    def fill_zero(j, carry):
        gat[0, j, :] = jnp.zeros((16,), jnp.float32)
        return carry

    lax.fori_loop(0, CH, fill_zero, 0)
    for r in range(15):
        pltpu.sync_copy(
            gat.at[0],
            acc.at[pl.ds(s * ROWS_PER_TEC + r * CH, CH)])
    pltpu.sync_copy(
        gat.at[0, pl.ds(0, 256)],
        acc.at[pl.ds(s * ROWS_PER_TEC + 15 * CH, 256)])
    plsc.subcore_barrier()

    base0 = s * PER_TEC

    def start_idx(g, b):
        pltpu.async_copy(coli.at[pl.ds(base0 + g * CH, CH)], colv.at[b], semi)
        pltpu.async_copy(rowi.at[pl.ds(base0 + g * CH, CH)], rowv.at[b], semi)

    def wait_idx(b):
        pltpu.make_async_copy(coli.at[pl.ds(base0, CH)], colv.at[b], semi).wait()
        pltpu.make_async_copy(rowi.at[pl.ds(base0, CH)], rowv.at[b], semi).wait()

    def start_gather(b4, b3):
        pltpu.async_copy(xcat.at[c].at[colv.at[b4]], gat.at[b3], semg.at[b3])

    def wait_gather(b4, b3):
        pltpu.make_async_copy(xcat.at[c].at[colv.at[b4]], gat.at[b3],
                              semg.at[b3]).wait()

    def start_scatter(b3, b4, ps):
        pltpu.async_copy(gat.at[b3], acc.at[rowv.at[b4]], sems.at[ps],
                         add=True)

    def wait_scatter(b3, b4, ps):
        pltpu.make_async_copy(gat.at[b3], acc.at[rowv.at[b4]],
                              sems.at[ps]).wait()

    start_idx(0, 0)
    start_idx(1, 1)
    wait_idx(0)
    start_gather(0, 0)
    wait_idx(1)
    start_gather(1, 1)
    start_idx(2, 2)

    def body(g, carry):
        g0 = lax.rem(g, 3)
        g2 = lax.rem(g + 2, 3)
        i2 = lax.rem(g + 2, 4)
        i3 = lax.rem(g + 3, 4)
        ps = lax.rem(g, 2)

        @pl.when(g >= 2)
        def _():
            wait_scatter(g2, lax.rem(g + 2, 4), ps)

        @pl.when(g + 2 < N_CH)
        def _():
            wait_idx(i2)
            start_gather(i2, g2)

        wait_gather(lax.rem(g, 4), g0)
        start_scatter(g0, lax.rem(g, 4), ps)

        @pl.when(g + 3 < N_CH)
        def _():
            start_idx(g + 3, i3)

        return carry

    lax.fori_loop(0, N_CH, body, 0)
    wait_scatter(lax.rem(N_CH - 2, 3), lax.rem(N_CH - 2, 4),
                 lax.rem(N_CH - 2, 2))
    wait_scatter(lax.rem(N_CH - 1, 3), lax.rem(N_CH - 1, 4),
                 lax.rem(N_CH - 1, 2))
    plsc.subcore_barrier()
    pltpu.sync_copy(acc.at[slab], out.at[c, slab])


@functools.partial(
    pl.kernel,
    out_type=jax.ShapeDtypeStruct((NC, NNP, KH), jnp.float32),
    mesh=_mesh,
    scratch_types=[
        pltpu.VMEM((CH_D,), jnp.int32),
        pltpu.VMEM((CH_D, KH), jnp.float32),
        pltpu.VMEM_SHARED((NNP, KH), jnp.float32),
    ],
    compiler_params=pltpu.CompilerParams(use_tc_tiling_on_sc=False),
)
def _sc_deg(rowi, out, rowv, onev, acc):
    c = lax.axis_index("c")
    s = lax.axis_index("s")
    slab = pl.ds(s * ROWS_PER_TEC, ROWS_PER_TEC)

    def fill(j, val):
        onev[j, :] = jnp.full((16,), val, jnp.float32)
        return val

    lax.fori_loop(0, CH_D, lambda j, v: fill(j, 0.0), 0.0)
    for r in range(6):
        pltpu.sync_copy(
            onev, acc.at[pl.ds(s * ROWS_PER_TEC + r * CH_D, CH_D)])
    pltpu.sync_copy(
        onev.at[pl.ds(0, 256)],
        acc.at[pl.ds(s * ROWS_PER_TEC + 6 * CH_D, 256)])
    lax.fori_loop(0, CH_D, lambda j, v: fill(j, 1.0), 1.0)
    plsc.subcore_barrier()

    def body(g, carry):
        base = (c * NS + s) * PER_TEC_D + g * CH_D
        pltpu.sync_copy(rowi.at[pl.ds(base, CH_D)], rowv)
        pltpu.sync_copy(onev, acc.at[rowv], add=True)
        return carry

    lax.fori_loop(0, N_CH_D, body, 0)
    plsc.subcore_barrier()
    pltpu.sync_copy(acc.at[slab], out.at[c, slab])


BLK_P = 3128
GRID = P // BLK_P


def _tc_body(xh, agg, degp, bd, out):
    invd = 1.0 / (degp[0] + degp[1] + 1e-7)
    hl = 2.0 * xh[0] - agg[0] * invd
    hr = 2.0 * xh[1] - agg[1] * invd
    yl = jax.nn.sigmoid(
        jnp.dot(hl, bd[0, 0], preferred_element_type=jnp.float32)
        + jnp.dot(hr, bd[1, 0], preferred_element_type=jnp.float32))
    yr = jax.nn.sigmoid(
        jnp.dot(hl, bd[0, 1], preferred_element_type=jnp.float32)
        + jnp.dot(hr, bd[1, 1], preferred_element_type=jnp.float32))
    out[0] = yl
    out[1] = yr


def _tc_dense(xh, agg, degp, bd):
    spec3 = pl.BlockSpec((NC, BLK_P, 128), lambda i: (0, i, 0))
    return pl.pallas_call(
        _tc_body,
        out_shape=jax.ShapeDtypeStruct((NC, P, 128), jnp.float32),
        grid=(GRID,),
        in_specs=[spec3, spec3, spec3,
                  pl.BlockSpec((2, 2, 128, 128), lambda i: (0, 0, 0, 0))],
        out_specs=spec3,
    )(xh, agg, degp, bd)


def kernel(Gu, Gi, filters, edge_index):
    rowi = edge_index[0]
    coli = edge_index[1]
    x0 = jnp.concatenate(
        [Gu, Gi, jnp.zeros((NNP - NN, K), jnp.float32)], axis=0)
    yp = jnp.stack([x0[:, :KH].reshape(P, 128),
                    x0[:, KH:].reshape(P, 128)])

    eye8 = jnp.eye(8, dtype=jnp.float32)
    bds = jnp.stack([
        jnp.stack([
            jnp.stack([jnp.kron(eye8, filters[k][:KH, :KH]),
                       jnp.kron(eye8, filters[k][:KH, KH:])]),
            jnp.stack([jnp.kron(eye8, filters[k][KH:, :KH]),
                       jnp.kron(eye8, filters[k][KH:, KH:])]),
        ]) for k in range(NL)])

    degp = _sc_deg(rowi).reshape(NC, P, 128)

    pieces = [yp]
    for k in range(NL):
        agg = _sc_spmm(yp.reshape(NC, NNP, KH), coli,
                       rowi).reshape(NC, P, 128)
        yp = _tc_dense(yp, agg, degp, bds[k])
        pieces.append(yp)

    emb = jnp.concatenate(
        [p[h].reshape(NNP, KH) for p in pieces for h in (0, 1)], axis=1)
    return emb[:NU], emb[NU:NN]


# speedup vs baseline: 1.0577x; 1.0577x over previous
import functools

import jax
import jax.numpy as jnp
from jax import lax
from jax.experimental import pallas as pl
from jax.experimental.pallas import tpu as pltpu
from jax.experimental.pallas import tpu_sc as plsc

NU = 50000
NI = 50000
NN = NU + NI
NNP = 100096
P = NNP // 8
EDG = 1600000
K = 32
KH = 16
NL = 3

NC = 2
NS = 16

CH = 800
PER_TEC = EDG // NS
N_CH = PER_TEC // CH
ROWS_PER_TEC = NNP // NS

CH_D = 1000
PER_TEC_D = EDG // (NC * NS)
N_CH_D = PER_TEC_D // CH_D

_mesh = plsc.VectorSubcoreMesh(core_axis_name="c", subcore_axis_name="s")


@functools.partial(
    pl.kernel,
    out_type=jax.ShapeDtypeStruct((NC, NNP, KH), jnp.float32),
    mesh=_mesh,
    scratch_types=[
        pltpu.VMEM((3, CH), jnp.int32),
        pltpu.VMEM((3, CH), jnp.int32),
        pltpu.VMEM((2, CH, KH), jnp.float32),
        pltpu.VMEM_SHARED((NNP, KH), jnp.float32),
        pltpu.SemaphoreType.DMA,
        pltpu.SemaphoreType.DMA((2,)),
        pltpu.SemaphoreType.DMA,
    ],
    compiler_params=pltpu.CompilerParams(use_tc_tiling_on_sc=False),
)
def _sc_spmm(xcat, coli, rowi, out, colv, rowv, gat, acc, semi, semg, sems):
    c = lax.axis_index("c")
    s = lax.axis_index("s")
    slab = pl.ds(s * ROWS_PER_TEC, ROWS_PER_TEC)

    def fill_zero(j, carry):
        gat[0, j, :] = jnp.zeros((16,), jnp.float32)
        return carry

    lax.fori_loop(0, CH, fill_zero, 0)
    for r in range(7):
        pltpu.sync_copy(
            gat.at[0],
            acc.at[pl.ds(s * ROWS_PER_TEC + r * CH, CH)])
    pltpu.sync_copy(
        gat.at[0, pl.ds(0, 656)],
        acc.at[pl.ds(s * ROWS_PER_TEC + 7 * CH, 656)])
    plsc.subcore_barrier()

    base0 = s * PER_TEC

    def start_idx(g, b):
        pltpu.async_copy(coli.at[pl.ds(base0 + g * CH, CH)], colv.at[b], semi)
        pltpu.async_copy(rowi.at[pl.ds(base0 + g * CH, CH)], rowv.at[b], semi)

    def wait_idx(b):
        pltpu.make_async_copy(coli.at[pl.ds(base0, CH)], colv.at[b], semi).wait()
        pltpu.make_async_copy(rowi.at[pl.ds(base0, CH)], rowv.at[b], semi).wait()

    def start_gather(b3, b2):
        pltpu.async_copy(xcat.at[c].at[colv.at[b3]], gat.at[b2], semg.at[b2])

    def wait_gather(b3, b2):
        pltpu.make_async_copy(xcat.at[c].at[colv.at[b3]], gat.at[b2],
                              semg.at[b2]).wait()

    def start_scatter(b2, b3):
        pltpu.async_copy(gat.at[b2], acc.at[rowv.at[b3]], sems, add=True)

    def wait_scatter(b2, b3):
        pltpu.make_async_copy(gat.at[b2], acc.at[rowv.at[b3]], sems).wait()

    start_idx(0, 0)
    wait_idx(0)
    start_idx(1, 1)
    start_gather(0, 0)

    def body(g, carry):
        pg = lax.rem(g, 2)
        qg = 1 - pg
        b0 = lax.rem(g, 3)
        b1 = lax.rem(g + 1, 3)
        b2 = lax.rem(g + 2, 3)

        @pl.when(g >= 1)
        def _():
            wait_scatter(qg, b2)

        @pl.when(g + 1 < N_CH)
        def _():
            wait_idx(b1)
            start_gather(b1, qg)

        wait_gather(b0, pg)
        start_scatter(pg, b0)

        @pl.when(g + 2 < N_CH)
        def _():
            start_idx(g + 2, b2)

        return carry

    lax.fori_loop(0, N_CH, body, 0)
    wait_scatter(lax.rem(N_CH - 1, 2), lax.rem(N_CH - 1, 3))
    plsc.subcore_barrier()
    pltpu.sync_copy(acc.at[slab], out.at[c, slab])


@functools.partial(
    pl.kernel,
    out_type=jax.ShapeDtypeStruct((NC, NNP, KH), jnp.float32),
    mesh=_mesh,
    scratch_types=[
        pltpu.VMEM((4, CH_D), jnp.int32),
        pltpu.VMEM((CH_D, KH), jnp.float32),
        pltpu.VMEM_SHARED((NNP, KH), jnp.float32),
        pltpu.SemaphoreType.DMA,
        pltpu.SemaphoreType.DMA((2,)),
    ],
    compiler_params=pltpu.CompilerParams(use_tc_tiling_on_sc=False),
)
def _sc_deg(rowi, out, rowv, onev, acc, semi, sems):
    c = lax.axis_index("c")
    s = lax.axis_index("s")
    slab = pl.ds(s * ROWS_PER_TEC, ROWS_PER_TEC)

    def fill(j, val):
        onev[j, :] = jnp.full((16,), val, jnp.float32)
        return val

    lax.fori_loop(0, CH_D, lambda j, v: fill(j, 0.0), 0.0)
    for r in range(6):
        pltpu.sync_copy(
            onev, acc.at[pl.ds(s * ROWS_PER_TEC + r * CH_D, CH_D)])
    pltpu.sync_copy(
        onev.at[pl.ds(0, 256)],
        acc.at[pl.ds(s * ROWS_PER_TEC + 6 * CH_D, 256)])
    lax.fori_loop(0, CH_D, lambda j, v: fill(j, 1.0), 1.0)
    plsc.subcore_barrier()

    base0 = (c * NS + s) * PER_TEC_D

    def start_idx(g, b):
        pltpu.async_copy(rowi.at[pl.ds(base0 + g * CH_D, CH_D)],
                         rowv.at[b], semi)

    def wait_idx(b):
        pltpu.make_async_copy(rowi.at[pl.ds(base0, CH_D)],
                              rowv.at[b], semi).wait()

    def start_scatter(b, p):
        pltpu.async_copy(onev, acc.at[rowv.at[b]], sems.at[p], add=True)

    def wait_scatter(b, p):
        pltpu.make_async_copy(onev, acc.at[rowv.at[b]], sems.at[p]).wait()

    start_idx(0, 0)
    start_idx(1, 1)

    def body(g, carry):
        p = lax.rem(g, 2)
        b0 = lax.rem(g, 4)
        b2 = lax.rem(g + 2, 4)

        @pl.when(g >= 2)
        def _():
            wait_scatter(b2, p)

        wait_idx(b0)
        start_scatter(b0, p)

        @pl.when(g + 2 < N_CH_D)
        def _():
            start_idx(g + 2, b2)

        return carry

    lax.fori_loop(0, N_CH_D, body, 0)
    wait_scatter(lax.rem(N_CH_D - 2, 4), lax.rem(N_CH_D - 2, 2))
    wait_scatter(lax.rem(N_CH_D - 1, 4), lax.rem(N_CH_D - 1, 2))
    plsc.subcore_barrier()
    pltpu.sync_copy(acc.at[slab], out.at[c, slab])


BLK_P = 3128
GRID = P // BLK_P


def _tc_body(xh, agg, degp, bd, out):
    invd = 1.0 / (degp[0] + degp[1] + 1e-7)
    hl = 2.0 * xh[0] - agg[0] * invd
    hr = 2.0 * xh[1] - agg[1] * invd
    yl = jax.nn.sigmoid(
        jnp.dot(hl, bd[0, 0], preferred_element_type=jnp.float32)
        + jnp.dot(hr, bd[1, 0], preferred_element_type=jnp.float32))
    yr = jax.nn.sigmoid(
        jnp.dot(hl, bd[0, 1], preferred_element_type=jnp.float32)
        + jnp.dot(hr, bd[1, 1], preferred_element_type=jnp.float32))
    out[0] = yl
    out[1] = yr


def _tc_dense(xh, agg, degp, bd):
    spec3 = pl.BlockSpec((NC, BLK_P, 128), lambda i: (0, i, 0))
    return pl.pallas_call(
        _tc_body,
        out_shape=jax.ShapeDtypeStruct((NC, P, 128), jnp.float32),
        grid=(GRID,),
        in_specs=[spec3, spec3, spec3,
                  pl.BlockSpec((2, 2, 128, 128), lambda i: (0, 0, 0, 0))],
        out_specs=spec3,
    )(xh, agg, degp, bd)


def kernel(Gu, Gi, filters, edge_index):
    rowi = edge_index[0]
    coli = edge_index[1]
    x0 = jnp.concatenate(
        [Gu, Gi, jnp.zeros((NNP - NN, K), jnp.float32)], axis=0)
    yp = jnp.stack([x0[:, :KH].reshape(P, 128),
                    x0[:, KH:].reshape(P, 128)])

    eye8 = jnp.eye(8, dtype=jnp.float32)
    bds = jnp.stack([
        jnp.stack([
            jnp.stack([jnp.kron(eye8, filters[k][:KH, :KH]),
                       jnp.kron(eye8, filters[k][:KH, KH:])]),
            jnp.stack([jnp.kron(eye8, filters[k][KH:, :KH]),
                       jnp.kron(eye8, filters[k][KH:, KH:])]),
        ]) for k in range(NL)])

    degp = _sc_deg(rowi).reshape(NC, P, 128)

    pieces = [yp]
    for k in range(NL):
        agg = _sc_spmm(yp.reshape(NC, NNP, KH), coli,
                       rowi).reshape(NC, P, 128)
        yp = _tc_dense(yp, agg, degp, bds[k])
        pieces.append(yp)

    emb = jnp.concatenate(
        [p[h].reshape(NNP, KH) for p in pieces for h in (0, 1)], axis=1)
    return emb[:NU], emb[NU:NN]
